# Initial kernel scaffold; baseline (speedup 1.0000x reference)
import jax
import jax.numpy as jnp
from jax.experimental import pallas as pl


def kernel(e_conv0_w, e_bn0_g, e_bn0_b, e_conv1_w, e_bn1_g, e_bn1_b, e_conv2_w, e_bn2_g, e_bn2_b, e_conv3_w, e_bn3_g, e_bn3_b, e_fc1_w, e_bn4_g, e_bn4_b, e_fc2_wmu, e_fc2_wsig, e_fc2_bmu, e_fc2_bsig, d_fc_w, d_bn0_g, d_bn0_b, d_tconv1_w, d_bn1_g, d_bn1_b, d_tconv2_w, d_bn2_g, d_bn2_b, d_tconv3_w, d_bn3_g, d_bn3_b, d_out_w, d_out_b, x, eps):
    raise NotImplementedError("write your pallas kernel here")


# speedup vs baseline: 37.7955x; 37.7955x over previous
import jax
import jax.numpy as jnp
from jax.experimental import pallas as pl
from jax.experimental.pallas import tpu as pltpu

_VMEM_LIMIT = 60 * 1024 * 1024


def _silu(y):
    return y * jax.nn.sigmoid(y)


def _bn_silu(y, g, b, phases=1):
    m = y.shape[0]
    s1 = jnp.sum(y, axis=0, keepdims=True)
    s2 = jnp.sum(y * y, axis=0, keepdims=True)
    if phases > 1:
        c = y.shape[1] // phases
        s1 = sum(s1[:, p * c:(p + 1) * c] for p in range(phases))
        s2 = sum(s2[:, p * c:(p + 1) * c] for p in range(phases))
    inv_n = 1.0 / (m * phases)
    mean = s1 * inv_n
    var = jnp.maximum(s2 * inv_n - mean * mean, 0.0)
    scale = g * jax.lax.rsqrt(var + 1e-5)
    shift = b - mean * scale
    if phases > 1:
        scale = jnp.concatenate([scale] * phases, axis=1)
        shift = jnp.concatenate([shift] * phases, axis=1)
    yh = y * scale + shift
    return _silu(yh)


def _pad_hw(x):
    n, h, w, c = x.shape
    zw = jnp.zeros((n, h, 1, c), x.dtype)
    x = jnp.concatenate([zw, x, zw], axis=2)
    zh = jnp.zeros((n, 1, w + 2, c), x.dtype)
    return jnp.concatenate([zh, x, zh], axis=1)


def _conv_mm(x, w_ref, stride):
    assert stride == 1
    n, h, w, c = x.shape
    m = n * h * w
    xp = _pad_hw(x)

    def tap(i, j):
        return xp[:, i:i + h, j:j + w, :]

    idx = [(i, j) for i in range(3) for j in range(3)]
    g = max(1, 256 // c)
    cout = w_ref.shape[1]
    y = jnp.zeros((m, cout), jnp.float32)
    for s in range(0, 9, g):
        chunk = idx[s:s + g]
        if len(chunk) == 1:
            blk = tap(*chunk[0]).reshape(m, c)
        else:
            blk = jnp.concatenate([tap(i, j) for i, j in chunk],
                                  axis=-1).reshape(m, len(chunk) * c)
        wblk = w_ref[s * c:(s + len(chunk)) * c, :]
        y = y + jnp.dot(blk, wblk, preferred_element_type=jnp.float32)
    return y


def _packed_s2_conv(a, w_ref, c):
    n, hh, v, _ = a.shape
    ho = hh // 2
    cout = w_ref.shape[1]
    m = n * ho * v

    ap = a.reshape(n, ho, 2, v, 8 * c)
    ae, ao = ap[:, :, 0], ap[:, :, 1]
    zrow = jnp.zeros((n, 1, v, 8 * c), a.dtype)
    ao_m1 = jnp.concatenate([zrow, ao[:, :ho - 1]], axis=1)
    rows = {0: ao_m1, 1: ae, 2: ao}

    def tap(i, j):
        r = rows[i]
        if j == 0:
            if v == 1:
                rp = jnp.zeros((n, ho, 1, 8 * c), a.dtype)
            else:
                zv = jnp.zeros((n, ho, 1, 8 * c), a.dtype)
                rp = jnp.concatenate([zv, r[:, :, :v - 1, :]], axis=2)
            parts = [rp[..., 7 * c:8 * c], r[..., c:2 * c],
                     r[..., 3 * c:4 * c], r[..., 5 * c:6 * c]]
        elif j == 1:
            parts = [r[..., 0:c], r[..., 2 * c:3 * c],
                     r[..., 4 * c:5 * c], r[..., 6 * c:7 * c]]
        else:
            parts = [r[..., c:2 * c], r[..., 3 * c:4 * c],
                     r[..., 5 * c:6 * c], r[..., 7 * c:8 * c]]
        return jnp.concatenate(parts, axis=-1).reshape(m, 4 * c)

    zc = jnp.zeros((c, cout), w_ref.dtype)

    def bdiag(t):
        wt = w_ref[t * c:(t + 1) * c, :]
        return jnp.concatenate(
            [jnp.concatenate([wt if k == q else zc for k in range(4)], axis=1)
             for q in range(4)], axis=0)

    idx = [(i, j) for i in range(3) for j in range(3)]
    y = jnp.zeros((m, 4 * cout), jnp.float32)
    for s in range(0, 9, 2):
        chunk = idx[s:s + 2]
        if len(chunk) == 2:
            blk = jnp.concatenate([tap(i, j) for i, j in chunk], axis=-1)
            wblk = jnp.concatenate([bdiag(s), bdiag(s + 1)], axis=0)
        else:
            blk = tap(*chunk[0])
            wblk = bdiag(s)
        y = y + jnp.dot(blk, wblk, preferred_element_type=jnp.float32)
    return y


def _repack8(a):
    n, hh, v, l = a.shape
    ar = a.reshape(n, hh, v // 2, 2, l)
    return jnp.concatenate([ar[:, :, :, 0, :], ar[:, :, :, 1, :]], axis=-1)


def _depth_to_space(y, c):
    n, h, w, _ = y.shape
    r0 = y[..., :2 * c].reshape(n, h, 1, w, 2 * c)
    r1 = y[..., 2 * c:].reshape(n, h, 1, w, 2 * c)
    t = jnp.concatenate([r0, r1], axis=2).reshape(n, 2 * h, w, 2 * c)
    u0 = t[..., :c].reshape(n, 2 * h, w, 1, c)
    u1 = t[..., c:].reshape(n, 2 * h, w, 1, c)
    return jnp.concatenate([u0, u1], axis=3).reshape(n, 2 * h, 2 * w, c)


def _enc_body(x_ref, eps_ref,
              w0_ref, g0_ref, b0_ref,
              w1_ref, g1_ref, b1_ref,
              w2_ref, g2_ref, b2_ref,
              w3_ref, g3_ref, b3_ref,
              wfc1_ref, g4_ref, b4_ref,
              wmu_ref, wsig_ref, bmu_ref, bsig_ref,
              z_ref, mu_ref, sig_ref):
    n = eps_ref.shape[0]

    w0 = w0_ref[...]
    zw = jnp.zeros((3, 32), jnp.bfloat16)
    w0d = jnp.concatenate(
        [jnp.concatenate([w0 if k == p else zw for k in range(8)], axis=1)
         for p in range(8)], axis=0)
    y0 = jnp.dot(x_ref[...], w0d, preferred_element_type=jnp.float32)
    h0p = _bn_silu(y0, g0_ref[...], b0_ref[...], phases=8).astype(jnp.bfloat16)
    a0 = h0p.reshape(n, 32, 4, 256)

    y1 = _packed_s2_conv(a0, w1_ref, 32)
    h1 = _bn_silu(y1, g1_ref[...], b1_ref[...], phases=4).astype(jnp.bfloat16)
    a1 = _repack8(h1.reshape(n, 16, 4, 128))

    y2 = _packed_s2_conv(a1, w2_ref, 32)
    h2 = _bn_silu(y2, g2_ref[...], b2_ref[...], phases=4).astype(jnp.bfloat16)
    a2 = _repack8(h2.reshape(n, 8, 2, 256))

    y3 = _packed_s2_conv(a2, w3_ref, 64)
    h3 = _bn_silu(y3, g3_ref[...], b3_ref[...], phases=4).astype(jnp.bfloat16)
    h3p = h3.reshape(n, 4, 1, 512)

    acc = jnp.zeros((n, 1024), jnp.float32)
    for q in range(16):
        a = h3p[:, q // 4, 0, 128 * (q % 4):128 * (q % 4 + 1)]
        acc = acc + jnp.dot(a, wfc1_ref[128 * q:128 * (q + 1), :],
                            preferred_element_type=jnp.float32)
    h4 = _bn_silu(acc, g4_ref[...], b4_ref[...]).astype(jnp.bfloat16)

    mu = jnp.dot(h4, wmu_ref[...], preferred_element_type=jnp.float32) + bmu_ref[...]
    ls = jnp.dot(h4, wsig_ref[...], preferred_element_type=jnp.float32) + bsig_ref[...]
    sig = jnp.exp(ls) + 1e-5
    mu_ref[...] = mu
    sig_ref[...] = sig
    z_ref[...] = mu + sig * eps_ref[...]


def _dec_body(z_ref,
              wdfc_ref, gd0_ref, bd0_ref,
              wt1_ref, gd1_ref, bd1_ref,
              wt2_ref, gd2_ref, bd2_ref,
              wt3_ref, gd3_ref, bd3_ref,
              wout_ref, bout_ref,
              y_ref):
    n = z_ref.shape[0]

    y5 = jnp.dot(z_ref[...].astype(jnp.bfloat16), wdfc_ref[...],
                 preferred_element_type=jnp.float32)
    h5 = _bn_silu(y5, gd0_ref[...], bd0_ref[...]).astype(jnp.bfloat16)
    parts = [h5[:, 512 * q:512 * (q + 1)].reshape(n, 1, 512) for q in range(16)]
    h5 = jnp.concatenate(parts, axis=1).reshape(n, 4, 4, 512)

    y6 = _conv_mm(h5, wt1_ref, 1)
    h6 = _bn_silu(y6, gd1_ref[...], bd1_ref[...], phases=4).astype(jnp.bfloat16)
    h6 = _depth_to_space(h6.reshape(n, 4, 4, 512), 128)

    y7 = _conv_mm(h6, wt2_ref, 1)
    h7 = _bn_silu(y7, gd2_ref[...], bd2_ref[...], phases=4).astype(jnp.bfloat16)
    h7 = _depth_to_space(h7.reshape(n, 8, 8, 256), 64)

    y8 = _conv_mm(h7, wt3_ref, 1)
    h8 = _bn_silu(y8, gd3_ref[...], bd3_ref[...], phases=4)
    out = jnp.dot(h8, wout_ref[...],
                  preferred_element_type=jnp.float32) + bout_ref[...]
    y_ref[...] = jnp.tanh(out)


def kernel(e_conv0_w, e_bn0_g, e_bn0_b, e_conv1_w, e_bn1_g, e_bn1_b,
           e_conv2_w, e_bn2_g, e_bn2_b, e_conv3_w, e_bn3_g, e_bn3_b,
           e_fc1_w, e_bn4_g, e_bn4_b, e_fc2_wmu, e_fc2_wsig, e_fc2_bmu,
           e_fc2_bsig, d_fc_w, d_bn0_g, d_bn0_b, d_tconv1_w, d_bn1_g,
           d_bn1_b, d_tconv2_w, d_bn2_g, d_bn2_b, d_tconv3_w, d_bn3_g,
           d_bn3_b, d_out_w, d_out_b, x, eps):
    n = x.shape[0]
    zdim = eps.shape[1]
    xr = (x.transpose(0, 2, 3, 1).reshape(n * 128, 24).astype(jnp.bfloat16))

    zms = jax.ShapeDtypeStruct((n, zdim), jnp.float32)
    z, mu, sig = pl.pallas_call(
        _enc_body,
        out_shape=(zms, zms, zms),
        compiler_params=pltpu.CompilerParams(vmem_limit_bytes=_VMEM_LIMIT),
    )(xr, eps,
      e_conv0_w, e_bn0_g, e_bn0_b,
      e_conv1_w, e_bn1_g, e_bn1_b,
      e_conv2_w, e_bn2_g, e_bn2_b,
      e_conv3_w, e_bn3_g, e_bn3_b,
      e_fc1_w, e_bn4_g, e_bn4_b,
      e_fc2_wmu, e_fc2_wsig, e_fc2_bmu, e_fc2_bsig)

    y = pl.pallas_call(
        _dec_body,
        out_shape=jax.ShapeDtypeStruct((n * 256, 12), jnp.float32),
        compiler_params=pltpu.CompilerParams(vmem_limit_bytes=_VMEM_LIMIT),
    )(z,
      d_fc_w, d_bn0_g, d_bn0_b,
      d_tconv1_w, d_bn1_g, d_bn1_b,
      d_tconv2_w, d_bn2_g, d_bn2_b,
      d_tconv3_w, d_bn3_g, d_bn3_b,
      d_out_w, d_out_b)

    x_hat = (y.reshape(n, 16, 16, 2, 2, 3).transpose(0, 1, 3, 2, 4, 5)
             .reshape(n, 32, 32, 3).transpose(0, 3, 1, 2))
    return z, mu, sig, x_hat
